# baseline (speedup 1.0000x reference)
import functools

import jax
import jax.numpy as jnp
from jax import lax
from jax.experimental import pallas as pl
from jax.experimental.pallas import tpu as pltpu
from jax.experimental.pallas import tpu_sc as plsc


def kernel(x):
    b, s, m, d = x.shape
    cr = m // 2
    nc = cr - 2
    P = b * s
    NW = 32
    per_w = P // NW
    CH = 4
    n_chunks = per_w // CH
    L = 16
    IW = m * d
    OW = nc * 6 * d

    xf = x.reshape(P * IW)

    mesh = plsc.VectorSubcoreMesh(core_axis_name="c", subcore_axis_name="s")

    @functools.partial(
        pl.kernel,
        mesh=mesh,
        out_type=jax.ShapeDtypeStruct((P * OW,), jnp.float32),
        scratch_types=[
            pltpu.VMEM((CH * IW,), jnp.float32),
            pltpu.VMEM((CH * IW,), jnp.float32),
            pltpu.VMEM((CH * OW,), jnp.float32),
            pltpu.VMEM((CH * OW,), jnp.float32),
            pltpu.SemaphoreType.DMA,
            pltpu.SemaphoreType.DMA,
            pltpu.SemaphoreType.DMA,
            pltpu.SemaphoreType.DMA,
        ],
    )
    def recombine(x_hbm, out_hbm, inb0, inb1, outb0, outb1,
                  sin0, sin1, sout0, sout1):
        inb = (inb0, inb1)
        outb = (outb0, outb1)
        sin = (sin0, sin1)
        sout = (sout0, sout1)

        cid = lax.axis_index("c")
        sid = lax.axis_index("s")
        wid = sid * 2 + cid
        base = wid * per_w

        def start_in(ci, slot):
            off = (base + ci * CH) * IW
            pltpu.async_copy(x_hbm.at[pl.ds(off, CH * IW)], inb[slot],
                             sin[slot])

        def wait_in(slot):
            pltpu.make_async_copy(
                x_hbm.at[pl.ds(base * IW, CH * IW)], inb[slot],
                sin[slot]).wait()

        def start_out(ci, slot):
            off = (base + ci * CH) * OW
            pltpu.async_copy(outb[slot], out_hbm.at[pl.ds(off, CH * OW)],
                             sout[slot])

        def wait_out(slot):
            pltpu.make_async_copy(
                outb[slot], out_hbm.at[pl.ds(base * OW, CH * OW)],
                sout[slot]).wait()

        def shuffle(slot):
            src = inb[slot]
            dst = outb[slot]
            for p in range(CH):
                halves = {}

                def get(r, h, _p=p):
                    key = (r, h)
                    if key not in halves:
                        halves[key] = src[pl.ds(_p * IW + r * d + h * L, L)]
                    return halves[key]

                for c in range(nc):
                    rows = (0, 1, 2 + c, cr, cr + 1, cr + 2 + c)
                    for j, r in enumerate(rows):
                        for h in range(d // L):
                            dst[pl.ds(p * OW + (c * 6 + j) * d + h * L, L)] \
                                = get(r, h)

        start_in(0, 0)
        start_in(1, 1)

        def body(g, carry):
            for slot in (0, 1):
                ci = 2 * g + slot
                wait_in(slot)

                @pl.when(g > 0)
                def _drain():
                    wait_out(slot)

                shuffle(slot)
                start_in(jnp.minimum(ci + 2, n_chunks - 1), slot)
                start_out(ci, slot)
            return carry

        lax.fori_loop(0, n_chunks // 2, body, 0)

        wait_in(0)
        wait_in(1)
        wait_out(0)
        wait_out(1)

    out = recombine(xf)
    return out.reshape(b, s, nc, 6, d)


# speedup vs baseline: 10.7207x; 10.7207x over previous
import functools

import jax
import jax.numpy as jnp
from jax import lax
from jax.experimental import pallas as pl
from jax.experimental.pallas import tpu as pltpu
from jax.experimental.pallas import tpu_sc as plsc


def kernel(x):
    b, s, m, d = x.shape
    cr = m // 2
    nc = cr - 2
    K = nc * 6
    HD = d // 2

    xp = jnp.transpose(x, (0, 2, 3, 1))

    mesh = plsc.VectorSubcoreMesh(core_axis_name="c", subcore_axis_name="s")

    @functools.partial(
        pl.kernel,
        mesh=mesh,
        out_type=jax.ShapeDtypeStruct((b, K, d, s), jnp.float32),
        scratch_types=[
            pltpu.VMEM((HD, s), jnp.float32),
            pltpu.VMEM((HD, s), jnp.float32),
            pltpu.VMEM((HD, s), jnp.float32),
            pltpu.SemaphoreType.DMA,
            pltpu.SemaphoreType.DMA,
            pltpu.SemaphoreType.DMA,
            pltpu.SemaphoreType.DMA,
            pltpu.SemaphoreType.DMA,
            pltpu.SemaphoreType.DMA,
        ],
        compiler_params=pltpu.CompilerParams(use_tc_tiling_on_sc=True),
    )
    def recombine(x_hbm, out_hbm, hbuf, lbuf0, lbuf1,
                  s_hr, s_hw, s_lr0, s_lr1, s_lw0, s_lw1):
        lbuf = (lbuf0, lbuf1)
        s_lr = (s_lr0, s_lr1)
        s_lw = (s_lw0, s_lw1)

        cid = lax.axis_index("c")
        sid = lax.axis_index("s")
        wid = sid * 2 + cid
        bb = wid // 8
        h8 = wid % 8

        t = h8 // 2
        hh = (h8 % 2) * HD
        r_h = t + jnp.where(t >= 2, m // 2 - 2, 0)
        j_h = t + jnp.where(t >= 2, 1, 0)

        def light_coords(q):
            li = h8 * 9 + q
            row_idx = li // 2
            lh = (li % 2) * HD
            r_l = row_idx + 2 + jnp.where(row_idx >= nc, 2, 0)
            k_l = jnp.where(row_idx < nc,
                            row_idx * 6 + 2, (row_idx - nc) * 6 + 5)
            return r_l, k_l, lh

        def rd(r, h, buf, sem):
            pltpu.async_copy(x_hbm.at[bb, r, pl.ds(h, HD), :], buf, sem)

        def wr(k, h, buf, sem):
            pltpu.async_copy(buf, out_hbm.at[bb, k, pl.ds(h, HD), :], sem)

        def wait_rd(buf, sem):
            pltpu.make_async_copy(
                x_hbm.at[bb, 0, pl.ds(0, HD), :], buf, sem).wait()

        def wait_wr(buf, sem):
            pltpu.make_async_copy(
                buf, out_hbm.at[bb, 0, pl.ds(0, HD), :], sem).wait()

        rd(r_h, hh, hbuf, s_hr)
        for q in (0, 1):
            r_l, _, lh = light_coords(q)
            rd(r_l, lh, lbuf[q], s_lr[q])

        wait_rd(hbuf, s_hr)
        for c in range(nc):
            wr(c * 6 + j_h, hh, hbuf, s_hw)

        for q in range(9):
            sl = q % 2
            r_l, k_l, lh = light_coords(q)
            wait_rd(lbuf[sl], s_lr[sl])
            wr(k_l, lh, lbuf[sl], s_lw[sl])
            if q + 2 <= 8:
                wait_wr(lbuf[sl], s_lw[sl])
                r_n, _, nh = light_coords(q + 2)
                rd(r_n, nh, lbuf[sl], s_lr[sl])

        for _ in range(nc):
            wait_wr(hbuf, s_hw)
        wait_wr(lbuf[0], s_lw[0])
        wait_wr(lbuf[1], s_lw[1])

    outp = recombine(xp)
    out = outp.reshape(b, nc, 6, d, s)
    return jnp.transpose(out, (0, 4, 1, 2, 3))


# speedup vs baseline: 10.9710x; 1.0233x over previous
import functools

import jax
import jax.numpy as jnp
from jax import lax
from jax.experimental import pallas as pl
from jax.experimental.pallas import tpu as pltpu
from jax.experimental.pallas import tpu_sc as plsc


def kernel(x):
    b, s, m, d = x.shape
    cr = m // 2
    nc = cr - 2
    K = nc * 6
    HD = d // 2
    QD = d // 4
    NLQ = 18
    NS = 5
    PF = 3

    xp = jnp.transpose(x, (0, 2, 3, 1))

    mesh = plsc.VectorSubcoreMesh(core_axis_name="c", subcore_axis_name="s")

    @functools.partial(
        pl.kernel,
        mesh=mesh,
        out_type=jax.ShapeDtypeStruct((b, K, d, s), jnp.float32),
        scratch_types=(
            [pltpu.VMEM((HD, s), jnp.float32)]
            + [pltpu.VMEM((QD, s), jnp.float32)] * NS
            + [pltpu.SemaphoreType.DMA] * (2 + 2 * NS)
        ),
        compiler_params=pltpu.CompilerParams(use_tc_tiling_on_sc=True),
    )
    def recombine(x_hbm, out_hbm, hbuf, *rest):
        lbuf = rest[:NS]
        s_hr, s_hw = rest[NS], rest[NS + 1]
        s_lr = rest[NS + 2:NS + 2 + NS]
        s_lw = rest[NS + 2 + NS:]

        cid = lax.axis_index("c")
        sid = lax.axis_index("s")
        wid = sid * 2 + cid
        bb = wid // 8
        h8 = wid % 8

        t = h8 // 2
        hh = (h8 % 2) * HD
        r_h = t + jnp.where(t >= 2, m // 2 - 2, 0)
        j_h = t + jnp.where(t >= 2, 1, 0)

        def light_coords(q):
            li = h8 * NLQ + q
            row_idx = li // 4
            lh = (li % 4) * QD
            r_l = row_idx + 2 + jnp.where(row_idx >= nc, 2, 0)
            k_l = jnp.where(row_idx < nc,
                            row_idx * 6 + 2, (row_idx - nc) * 6 + 5)
            return r_l, k_l, lh

        def rd(r, h, ht, buf, sem):
            pltpu.async_copy(x_hbm.at[bb, r, pl.ds(h, ht), :], buf, sem)

        def wr(k, h, ht, buf, sem):
            pltpu.async_copy(buf, out_hbm.at[bb, k, pl.ds(h, ht), :], sem)

        def wait_rd(ht, buf, sem):
            pltpu.make_async_copy(
                x_hbm.at[bb, 0, pl.ds(0, ht), :], buf, sem).wait()

        def wait_wr(ht, buf, sem):
            pltpu.make_async_copy(
                buf, out_hbm.at[bb, 0, pl.ds(0, ht), :], sem).wait()

        rd(r_h, hh, HD, hbuf, s_hr)
        for q in range(PF):
            r_l, _, lh = light_coords(q)
            rd(r_l, lh, QD, lbuf[q], s_lr[q])

        wait_rd(HD, hbuf, s_hr)
        for c in range(nc):
            wr(c * 6 + j_h, hh, HD, hbuf, s_hw)

        for q in range(NLQ):
            sl = q % NS
            r_l, k_l, lh = light_coords(q)
            wait_rd(QD, lbuf[sl], s_lr[sl])
            wr(k_l, lh, QD, lbuf[sl], s_lw[sl])
            i = q + PF
            if i < NLQ:
                si = i % NS
                if i - NS >= 0:
                    wait_wr(QD, lbuf[si], s_lw[si])
                r_n, _, nh = light_coords(i)
                rd(r_n, nh, QD, lbuf[si], s_lr[si])

        for _ in range(nc):
            wait_wr(HD, hbuf, s_hw)
        for i in range(NLQ - NS, NLQ):
            wait_wr(QD, lbuf[i % NS], s_lw[i % NS])

    outp = recombine(xp)
    out = outp.reshape(b, nc, 6, d, s)
    return jnp.transpose(out, (0, 4, 1, 2, 3))
